# baseline (speedup 1.0000x reference)
import functools

import jax
import jax.numpy as jnp
from jax import lax
from jax.experimental import pallas as pl
from jax.experimental.pallas import tpu as pltpu


def _pick_tile(dim, target):
    if dim <= target:
        return dim
    for t in range(target, 0, -1):
        if dim % t == 0:
            return t
    return dim


def _emb_gather_kernel(ids_ref, tab_ref, out_ref, *, rows, p, unroll):
    def outer(cc, carry):
        base = cc * unroll
        for u in range(unroll):
            idx = pl.multiple_of(ids_ref[0, 0, base + u], p)
            dst = pl.multiple_of((base + u) * p, p)
            out_ref[pl.ds(dst, p), :] = tab_ref[pl.ds(idx, p), :]
        return carry

    lax.fori_loop(0, rows // unroll, outer, 0)


def _embedding_lookup(table, ids):
    T, B = ids.shape
    ntoken, ninp = table.shape
    N = T * B
    p = ninp // 256
    tab_i32 = lax.bitcast_convert_type(
        table.reshape(ntoken, p, 128, 2), jnp.int32
    ).reshape(ntoken * p, 128)

    rows = 512 if N % 512 == 0 else _pick_tile(N, 512)
    G = N // rows
    unroll = 64 if rows % 64 == 0 else rows
    ids3d = (ids.reshape(1, N) * p).astype(jnp.int32).reshape(G, 1, rows)

    out = pl.pallas_call(
        functools.partial(_emb_gather_kernel, rows=rows, p=p, unroll=unroll),
        out_shape=jax.ShapeDtypeStruct((N * p, 128), jnp.int32),
        grid=(G,),
        in_specs=[
            pl.BlockSpec((1, 1, rows), lambda i: (i, 0, 0),
                         memory_space=pltpu.SMEM),
            pl.BlockSpec((ntoken * p, 128), lambda i: (0, 0)),
        ],
        out_specs=pl.BlockSpec((rows * p, 128), lambda i: (i, 0)),
        compiler_params=pltpu.CompilerParams(
            dimension_semantics=("parallel",),
            vmem_limit_bytes=int(ntoken * ninp * 2 * 2 + 8 * 1024 * 1024),
        ),
    )(ids3d, tab_i32)
    x = lax.bitcast_convert_type(
        out.reshape(N, p, 128), jnp.bfloat16
    ).reshape(T, B, ninp)
    return x


def _fused_lstm_kernel(x_ref, hc0_ref, w_ref, b_ref,
                       y_ref, hcT_ref, ybuf, gx,
                       *, tc, tb, H, nlayers, nchunks):
    l = pl.program_id(1)
    c = pl.program_id(2)

    @pl.when(c == 0)
    def _():
        hcT_ref[0, pl.ds(l, 1)] = hc0_ref[0, pl.ds(l, 1)]
        hcT_ref[1, pl.ds(l, 1)] = hc0_ref[1, pl.ds(l, 1)]

    w_ih = w_ref[0, 0]
    w_hh = w_ref[0, 1]
    bias = b_ref[0]

    @pl.when(l == 0)
    def _():
        x2d = x_ref[...].reshape(tc * tb, H)
        gx[...] = jnp.dot(x2d, w_ih, preferred_element_type=jnp.float32) + bias

    @pl.when(l > 0)
    def _():
        x2d = ybuf[pl.ds(c * tc, tc)].reshape(tc * tb, H)
        gx[...] = jnp.dot(x2d, w_ih, preferred_element_type=jnp.float32) + bias

    h = hcT_ref[0, pl.ds(l, 1)].reshape(tb, H)
    cs = hcT_ref[1, pl.ds(l, 1)].reshape(tb, H)
    for t in range(tc):
        gates = gx[pl.ds(t * tb, tb), :] + jnp.dot(
            h.astype(jnp.bfloat16), w_hh, preferred_element_type=jnp.float32)
        i_g = jax.nn.sigmoid(gates[:, 0 * H:1 * H])
        f_g = jax.nn.sigmoid(gates[:, 1 * H:2 * H])
        g_g = jnp.tanh(gates[:, 2 * H:3 * H])
        o_g = jax.nn.sigmoid(gates[:, 3 * H:4 * H])
        cs = f_g * cs + i_g * g_g
        h = o_g * jnp.tanh(cs)
        ybuf[pl.ds(c * tc + t, 1)] = h.astype(jnp.bfloat16).reshape(1, tb, H)
        y_ref[t] = h

    hcT_ref[0, pl.ds(l, 1)] = h.reshape(1, tb, H)
    hcT_ref[1, pl.ds(l, 1)] = cs.reshape(1, tb, H)


def _fused_lstm(x, h0, c0, w, b):
    T, B, H = x.shape
    L = w.shape[0]
    tc = _pick_tile(T, 8)
    tb = B // 2 if (B >= 16 and B % 16 == 0) else B
    nb = B // tb
    nchunks = T // tc
    hc0 = jnp.stack([h0, c0])

    last = L - 1
    kern = functools.partial(_fused_lstm_kernel, tc=tc, tb=tb, H=H,
                             nlayers=L, nchunks=nchunks)
    est = (2 * tc * tb * H * 2
           + 2 * 2 * H * 4 * H * 2
           + 2 * tc * tb * H * 4
           + 4 * L * tb * H * 4
           + T * tb * H * 2
           + tc * tb * 4 * H * 4
           + 12 * 1024 * 1024)
    y, hcT = pl.pallas_call(
        kern,
        out_shape=(
            jax.ShapeDtypeStruct((T, B, H), jnp.float32),
            jax.ShapeDtypeStruct((2, L, B, H), jnp.float32),
        ),
        grid=(nb, L, nchunks),
        in_specs=[
            pl.BlockSpec((tc, tb, H),
                         lambda bi, l, c: (jnp.where(l == 0, c, 0), bi, 0)),
            pl.BlockSpec((2, L, tb, H), lambda bi, l, c: (0, 0, bi, 0)),
            pl.BlockSpec((1, 2, H, 4 * H), lambda bi, l, c: (l, 0, 0, 0)),
            pl.BlockSpec((1, 1, 4 * H), lambda bi, l, c: (l, 0, 0)),
        ],
        out_specs=[
            pl.BlockSpec((tc, tb, H),
                         lambda bi, l, c: (jnp.where(l == last, c, 0), bi, 0)),
            pl.BlockSpec((2, L, tb, H), lambda bi, l, c: (0, 0, bi, 0)),
        ],
        scratch_shapes=[
            pltpu.VMEM((T, tb, H), jnp.bfloat16),
            pltpu.VMEM((tc * tb, 4 * H), jnp.float32),
        ],
        compiler_params=pltpu.CompilerParams(
            dimension_semantics=("parallel", "arbitrary", "arbitrary"),
            vmem_limit_bytes=int(min(est, 56 * 1024 * 1024)),
        ),
    )(x, hc0, w, b)
    return y, hcT[0], hcT[1]


def kernel(emb, input_ids, h0, c0,
           w_ih_0, w_hh_0, b_0,
           w_ih_1, w_hh_1, b_1,
           w_ih_2, w_hh_2, b_2,
           w_ih_3, w_hh_3, b_3):
    x = _embedding_lookup(emb, input_ids)
    w = jnp.stack([
        jnp.stack([w_ih_0, w_hh_0]),
        jnp.stack([w_ih_1, w_hh_1]),
        jnp.stack([w_ih_2, w_hh_2]),
        jnp.stack([w_ih_3, w_hh_3]),
    ])
    b = jnp.stack([b_0, b_1, b_2, b_3])
    y, hT, cT = _fused_lstm(x, h0, c0, w, b)
    return y, (hT, cT)


# speedup vs baseline: 1.0228x; 1.0228x over previous
import functools

import jax
import jax.numpy as jnp
from jax import lax
from jax.experimental import pallas as pl
from jax.experimental.pallas import tpu as pltpu


def _pick_tile(dim, target):
    if dim <= target:
        return dim
    for t in range(target, 0, -1):
        if dim % t == 0:
            return t
    return dim


def _emb_gather_kernel(ids_ref, tab_ref, out_ref, *, rows, p, unroll):
    def outer(cc, carry):
        base = cc * unroll
        for u in range(unroll):
            idx = pl.multiple_of(ids_ref[0, 0, base + u], p)
            dst = pl.multiple_of((base + u) * p, p)
            out_ref[pl.ds(dst, p), :] = tab_ref[pl.ds(idx, p), :]
        return carry

    lax.fori_loop(0, rows // unroll, outer, 0)


def _embedding_lookup(table, ids):
    T, B = ids.shape
    ntoken, ninp = table.shape
    N = T * B
    p = ninp // 256
    tab_i32 = lax.bitcast_convert_type(
        table.reshape(ntoken, p, 128, 2), jnp.int32
    ).reshape(ntoken * p, 128)

    rows = 512 if N % 512 == 0 else _pick_tile(N, 512)
    G = N // rows
    unroll = 64 if rows % 64 == 0 else rows
    ids3d = (ids.reshape(1, N) * p).astype(jnp.int32).reshape(G, 1, rows)

    out = pl.pallas_call(
        functools.partial(_emb_gather_kernel, rows=rows, p=p, unroll=unroll),
        out_shape=jax.ShapeDtypeStruct((N * p, 128), jnp.int32),
        grid=(G,),
        in_specs=[
            pl.BlockSpec((1, 1, rows), lambda i: (i, 0, 0),
                         memory_space=pltpu.SMEM),
            pl.BlockSpec((ntoken * p, 128), lambda i: (0, 0)),
        ],
        out_specs=pl.BlockSpec((rows * p, 128), lambda i: (i, 0)),
        compiler_params=pltpu.CompilerParams(
            dimension_semantics=("parallel",),
            vmem_limit_bytes=int(ntoken * ninp * 2 * 2 + 8 * 1024 * 1024),
        ),
    )(ids3d, tab_i32)
    x = lax.bitcast_convert_type(
        out.reshape(N, p, 128), jnp.bfloat16
    ).reshape(T, B, ninp)
    return x


def _sigmoid(x):
    return 0.5 * jnp.tanh(0.5 * x) + 0.5


def _fused_lstm_kernel(x_ref, h0_ref, c0_ref,
                       wi0, wi1, wi2, wi3, wh0, wh1, wh2, wh3, b_ref,
                       y_ref, hT_ref, cT_ref,
                       ybuf, gx, wih_s, whh_s, sem,
                       *, tc, tb, H, nlayers, nchunks):
    bi = pl.program_id(0)
    l = pl.program_id(1)
    c = pl.program_id(2)

    @pl.when((l == 0) & (c == 0))
    def _():
        cps = []
        for i, (wi, wh) in enumerate(((wi0, wh0), (wi1, wh1),
                                      (wi2, wh2), (wi3, wh3))):
            cps.append(pltpu.make_async_copy(wi, wih_s.at[i], sem))
            cps.append(pltpu.make_async_copy(wh, whh_s.at[i], sem))
        cps.append(pltpu.make_async_copy(
            h0_ref.at[:, pl.ds(bi * tb, tb)], hT_ref, sem))
        cps.append(pltpu.make_async_copy(
            c0_ref.at[:, pl.ds(bi * tb, tb)], cT_ref, sem))
        for cp in cps:
            cp.start()
        for cp in cps:
            cp.wait()

    bias = b_ref[pl.ds(l, 1)]
    w_ih = wih_s.at[l]
    w_hh = whh_s.at[l]

    @pl.when(l == 0)
    def _():
        x2d = x_ref[...].reshape(tc * tb, H)
        gx[...] = jnp.dot(x2d, w_ih[...],
                          preferred_element_type=jnp.float32) + bias

    @pl.when(l > 0)
    def _():
        x2d = ybuf[pl.ds(c * tc, tc)].reshape(tc * tb, H)
        gx[...] = jnp.dot(x2d, w_ih[...],
                          preferred_element_type=jnp.float32) + bias

    h = hT_ref[pl.ds(l, 1)].reshape(tb, H)
    cs = cT_ref[pl.ds(l, 1)].reshape(tb, H)
    for t in range(tc):
        hb = h.astype(jnp.bfloat16)
        row = pl.ds(t * tb, tb)
        i_g = _sigmoid(gx[row, 0 * H:1 * H] + jnp.dot(
            hb, whh_s[l, :, 0 * H:1 * H], preferred_element_type=jnp.float32))
        f_g = _sigmoid(gx[row, 1 * H:2 * H] + jnp.dot(
            hb, whh_s[l, :, 1 * H:2 * H], preferred_element_type=jnp.float32))
        g_g = jnp.tanh(gx[row, 2 * H:3 * H] + jnp.dot(
            hb, whh_s[l, :, 2 * H:3 * H], preferred_element_type=jnp.float32))
        o_g = _sigmoid(gx[row, 3 * H:4 * H] + jnp.dot(
            hb, whh_s[l, :, 3 * H:4 * H], preferred_element_type=jnp.float32))
        cs = f_g * cs + i_g * g_g
        h = o_g * jnp.tanh(cs)
        ybuf[pl.ds(c * tc + t, 1)] = h.astype(jnp.bfloat16).reshape(1, tb, H)
        y_ref[t] = h

    hT_ref[pl.ds(l, 1)] = h.reshape(1, tb, H)
    cT_ref[pl.ds(l, 1)] = cs.reshape(1, tb, H)


def _fused_lstm(x, h0, c0, wih, whh, bcat):
    T, B, H = x.shape
    L = len(wih)
    tc = _pick_tile(T, 8)
    tb = B // 2 if (B >= 16 and B % 16 == 0) else B
    nb = B // tb
    nchunks = T // tc

    last = L - 1
    kern = functools.partial(_fused_lstm_kernel, tc=tc, tb=tb, H=H,
                             nlayers=L, nchunks=nchunks)
    est = (2 * tc * tb * H * 2
           + 2 * L * H * 4 * H * 2
           + 2 * tc * tb * H * 4
           + 4 * L * tb * H * 4
           + T * tb * H * 2
           + tc * tb * 4 * H * 4
           + 12 * 1024 * 1024)
    any_spec = pl.BlockSpec(memory_space=pl.ANY)
    y, hT, cT = pl.pallas_call(
        kern,
        out_shape=(
            jax.ShapeDtypeStruct((T, B, H), jnp.float32),
            jax.ShapeDtypeStruct((L, B, H), jnp.float32),
            jax.ShapeDtypeStruct((L, B, H), jnp.float32),
        ),
        grid=(nb, L, nchunks),
        in_specs=[
            pl.BlockSpec((tc, tb, H),
                         lambda bi, l, c: (jnp.where(l == 0, c, 0), bi, 0)),
            any_spec, any_spec,
            any_spec, any_spec, any_spec, any_spec,
            any_spec, any_spec, any_spec, any_spec,
            pl.BlockSpec((L, 4 * H), lambda bi, l, c: (0, 0)),
        ],
        out_specs=[
            pl.BlockSpec((tc, tb, H),
                         lambda bi, l, c: (jnp.where(l == last, c, 0), bi, 0)),
            pl.BlockSpec((L, tb, H), lambda bi, l, c: (0, bi, 0)),
            pl.BlockSpec((L, tb, H), lambda bi, l, c: (0, bi, 0)),
        ],
        scratch_shapes=[
            pltpu.VMEM((T, tb, H), jnp.bfloat16),
            pltpu.VMEM((tc * tb, 4 * H), jnp.float32),
            pltpu.VMEM((L, H, 4 * H), jnp.bfloat16),
            pltpu.VMEM((L, H, 4 * H), jnp.bfloat16),
            pltpu.SemaphoreType.DMA,
        ],
        compiler_params=pltpu.CompilerParams(
            dimension_semantics=("parallel", "arbitrary", "arbitrary"),
            vmem_limit_bytes=int(min(est, 56 * 1024 * 1024)),
        ),
    )(x, h0, c0, *wih, *whh, bcat)
    return y, hT, cT


def kernel(emb, input_ids, h0, c0,
           w_ih_0, w_hh_0, b_0,
           w_ih_1, w_hh_1, b_1,
           w_ih_2, w_hh_2, b_2,
           w_ih_3, w_hh_3, b_3):
    x = _embedding_lookup(emb, input_ids)
    bcat = jnp.concatenate([b_0, b_1, b_2, b_3], axis=0)
    y, hT, cT = _fused_lstm(x, h0, c0,
                            (w_ih_0, w_ih_1, w_ih_2, w_ih_3),
                            (w_hh_0, w_hh_1, w_hh_2, w_hh_3), bcat)
    return y, (hT, cT)


# speedup vs baseline: 1.2924x; 1.2636x over previous
import functools

import jax
import jax.numpy as jnp
from jax import lax
from jax.experimental import pallas as pl
from jax.experimental.pallas import tpu as pltpu


def _pick_tile(dim, target):
    if dim <= target:
        return dim
    for t in range(target, 0, -1):
        if dim % t == 0:
            return t
    return dim


def _emb_gather_kernel(ids_ref, tab_ref, out_ref, *, unroll):
    rows = out_ref.shape[0]

    def outer(cc, carry):
        base = cc * unroll
        for u in range(unroll):
            idx = ids_ref[0, 0, base + u]
            out_ref[pl.ds(base + u, 1), 0, :] = tab_ref[pl.ds(idx, 1), 0, :]
        return carry

    lax.fori_loop(0, rows // unroll, outer, 0)


def _embedding_lookup(table, ids):
    T, B = ids.shape
    ntoken, ninp = table.shape
    N = T * B
    tab32 = table.astype(jnp.float32).reshape(ntoken, 1, ninp)

    rows = 512 if N % 512 == 0 else _pick_tile(N, 512)
    G = N // rows
    unroll = 64 if rows % 64 == 0 else rows
    ids3d = ids.reshape(G, 1, rows)

    out = pl.pallas_call(
        functools.partial(_emb_gather_kernel, unroll=unroll),
        out_shape=jax.ShapeDtypeStruct((N, 1, ninp), jnp.float32),
        grid=(G,),
        in_specs=[
            pl.BlockSpec((1, 1, rows), lambda i: (i, 0, 0),
                         memory_space=pltpu.SMEM),
            pl.BlockSpec((ntoken, 1, ninp), lambda i: (0, 0, 0)),
        ],
        out_specs=pl.BlockSpec((rows, 1, ninp), lambda i: (i, 0, 0)),
        compiler_params=pltpu.CompilerParams(
            dimension_semantics=("parallel",),
            vmem_limit_bytes=int(ntoken * ninp * 4 * 2 + 16 * 1024 * 1024),
        ),
    )(ids3d, tab32)
    return out.astype(jnp.bfloat16).reshape(T, B, ninp)


def _sigmoid(x):
    return 0.5 * jnp.tanh(0.5 * x) + 0.5


def _fused_lstm_kernel(x_ref, h0_ref, c0_ref,
                       wi0, wi1, wi2, wi3, wh0, wh1, wh2, wh3, b_ref,
                       y_ref, hT_ref, cT_ref,
                       ybuf, gx, wih_s, whh_s, sem,
                       *, tc, tb, H, nlayers, nchunks):
    bi = pl.program_id(0)
    l = pl.program_id(1)
    c = pl.program_id(2)

    @pl.when((l == 0) & (c == 0))
    def _():
        cps = []
        for i, (wi, wh) in enumerate(((wi0, wh0), (wi1, wh1),
                                      (wi2, wh2), (wi3, wh3))):
            cps.append(pltpu.make_async_copy(wi, wih_s.at[i], sem))
            cps.append(pltpu.make_async_copy(wh, whh_s.at[i], sem))
        cps.append(pltpu.make_async_copy(
            h0_ref.at[:, pl.ds(bi * tb, tb)], hT_ref, sem))
        cps.append(pltpu.make_async_copy(
            c0_ref.at[:, pl.ds(bi * tb, tb)], cT_ref, sem))
        for cp in cps:
            cp.start()
        for cp in cps:
            cp.wait()

    bias = b_ref[pl.ds(l, 1)]
    w_ih = wih_s.at[l]
    w_hh = whh_s.at[l]

    @pl.when(l == 0)
    def _():
        x2d = x_ref[...].reshape(tc * tb, H)
        gx[...] = jnp.dot(x2d, w_ih[...],
                          preferred_element_type=jnp.float32) + bias

    @pl.when(l > 0)
    def _():
        x2d = ybuf[pl.ds(c * tc, tc)].reshape(tc * tb, H)
        gx[...] = jnp.dot(x2d, w_ih[...],
                          preferred_element_type=jnp.float32) + bias

    h = hT_ref[pl.ds(l, 1)].reshape(tb, H)
    cs = cT_ref[pl.ds(l, 1)].reshape(tb, H)
    for t in range(tc):
        hb = h.astype(jnp.bfloat16)
        row = pl.ds(t * tb, tb)
        i_g = _sigmoid(gx[row, 0 * H:1 * H] + jnp.dot(
            hb, whh_s[l, :, 0 * H:1 * H], preferred_element_type=jnp.float32))
        f_g = _sigmoid(gx[row, 1 * H:2 * H] + jnp.dot(
            hb, whh_s[l, :, 1 * H:2 * H], preferred_element_type=jnp.float32))
        g_g = jnp.tanh(gx[row, 2 * H:3 * H] + jnp.dot(
            hb, whh_s[l, :, 2 * H:3 * H], preferred_element_type=jnp.float32))
        o_g = _sigmoid(gx[row, 3 * H:4 * H] + jnp.dot(
            hb, whh_s[l, :, 3 * H:4 * H], preferred_element_type=jnp.float32))
        cs = f_g * cs + i_g * g_g
        h = o_g * jnp.tanh(cs)
        ybuf[pl.ds(c * tc + t, 1)] = h.astype(jnp.bfloat16).reshape(1, tb, H)
        y_ref[t] = h

    hT_ref[pl.ds(l, 1)] = h.reshape(1, tb, H)
    cT_ref[pl.ds(l, 1)] = cs.reshape(1, tb, H)


def _fused_lstm(x, h0, c0, wih, whh, bcat):
    T, B, H = x.shape
    L = len(wih)
    tc = _pick_tile(T, 8)
    tb = B // 2 if (B >= 16 and B % 16 == 0) else B
    nb = B // tb
    nchunks = T // tc

    last = L - 1
    kern = functools.partial(_fused_lstm_kernel, tc=tc, tb=tb, H=H,
                             nlayers=L, nchunks=nchunks)
    est = (2 * tc * tb * H * 2
           + 2 * L * H * 4 * H * 2
           + 2 * tc * tb * H * 4
           + 4 * L * tb * H * 4
           + T * tb * H * 2
           + tc * tb * 4 * H * 4
           + 12 * 1024 * 1024)
    any_spec = pl.BlockSpec(memory_space=pl.ANY)
    y, hT, cT = pl.pallas_call(
        kern,
        out_shape=(
            jax.ShapeDtypeStruct((T, B, H), jnp.float32),
            jax.ShapeDtypeStruct((L, B, H), jnp.float32),
            jax.ShapeDtypeStruct((L, B, H), jnp.float32),
        ),
        grid=(nb, L, nchunks),
        in_specs=[
            pl.BlockSpec((tc, tb, H),
                         lambda bi, l, c: (jnp.where(l == 0, c, 0), bi, 0)),
            any_spec, any_spec,
            any_spec, any_spec, any_spec, any_spec,
            any_spec, any_spec, any_spec, any_spec,
            pl.BlockSpec((L, 4 * H), lambda bi, l, c: (0, 0)),
        ],
        out_specs=[
            pl.BlockSpec((tc, tb, H),
                         lambda bi, l, c: (jnp.where(l == last, c, 0), bi, 0)),
            pl.BlockSpec((L, tb, H), lambda bi, l, c: (0, bi, 0)),
            pl.BlockSpec((L, tb, H), lambda bi, l, c: (0, bi, 0)),
        ],
        scratch_shapes=[
            pltpu.VMEM((T, tb, H), jnp.bfloat16),
            pltpu.VMEM((tc * tb, 4 * H), jnp.float32),
            pltpu.VMEM((L, H, 4 * H), jnp.bfloat16),
            pltpu.VMEM((L, H, 4 * H), jnp.bfloat16),
            pltpu.SemaphoreType.DMA,
        ],
        compiler_params=pltpu.CompilerParams(
            dimension_semantics=("parallel", "arbitrary", "arbitrary"),
            vmem_limit_bytes=int(min(est, 56 * 1024 * 1024)),
        ),
    )(x, h0, c0, *wih, *whh, bcat)
    return y, hT, cT


def kernel(emb, input_ids, h0, c0,
           w_ih_0, w_hh_0, b_0,
           w_ih_1, w_hh_1, b_1,
           w_ih_2, w_hh_2, b_2,
           w_ih_3, w_hh_3, b_3):
    x = _embedding_lookup(emb, input_ids)
    bcat = jnp.concatenate([b_0, b_1, b_2, b_3], axis=0)
    y, hT, cT = _fused_lstm(x, h0, c0,
                            (w_ih_0, w_ih_1, w_ih_2, w_ih_3),
                            (w_hh_0, w_hh_1, w_hh_2, w_hh_3), bcat)
    return y, (hT, cT)
